# baseline (speedup 1.0000x reference)
import functools

import jax
import jax.numpy as jnp
from jax import lax
from jax.experimental import pallas as pl
from jax.experimental.pallas import tpu as pltpu
from jax.experimental.pallas import tpu_sc as plsc

B = 16384
NUM_FIELDS = 26
VOCAB = 100000
EMBED_DIM = 16
DENSE_DIM = 13
X0_DIM = 432
N_LOOKUPS = B * NUM_FIELDS

NC = 2
NS = 16
NW = NC * NS
PER_W = N_LOOKUPS // NW
CHUNK = 128
CHUNKS = PER_W // CHUNK
G = 8
STEPS = CHUNKS // G


def _gather_body(idx_hbm, table_hbm, out_hbm, idx_v, rows_v, sem):
    wid = lax.axis_index("s") * NC + lax.axis_index("c")
    pltpu.sync_copy(idx_hbm.at[wid], idx_v)
    base = wid * PER_W

    def step(t, carry):
        cps = []
        for g in range(G):
            cps.append(pltpu.async_copy(
                table_hbm.at[idx_v.at[t * G + g]],
                rows_v.at[pl.ds(g * CHUNK, CHUNK)],
                sem))
        for cp in cps:
            cp.wait()
        pltpu.sync_copy(rows_v, out_hbm.at[pl.ds(base + t * (G * CHUNK), G * CHUNK)])
        return carry

    lax.fori_loop(0, STEPS, step, 0)


def _sc_gather(idx3, table):
    k = functools.partial(
        pl.kernel,
        mesh=plsc.VectorSubcoreMesh(core_axis_name="c", subcore_axis_name="s"),
        out_type=jax.ShapeDtypeStruct((N_LOOKUPS, EMBED_DIM), jnp.float32),
        scratch_types=[
            pltpu.VMEM((CHUNKS, CHUNK), jnp.int32),
            pltpu.VMEM((G * CHUNK, EMBED_DIM), jnp.float32),
            pltpu.SemaphoreType.DMA,
        ],
        compiler_params=pltpu.CompilerParams(use_tc_tiling_on_sc=False),
    )(_gather_body)
    return k(idx3, table)


N_ROWS = NUM_FIELDS * VOCAB
TW = 16384
TBLK = pl.cdiv(N_ROWS, TW)
TC_ = TW // 8
PAD_PACKED = TBLK * TC_
PAD_ROWS = PAD_PACKED * 8


def _tr_body(in_ref, out_ref):
    eye = jnp.eye(EMBED_DIM, dtype=jnp.bfloat16)
    c = TW // 8
    for s in range(8):
        x = in_ref[:, s * c:(s + 1) * c]
        hi = x.astype(jnp.bfloat16)
        lo = (x - hi.astype(jnp.float32)).astype(jnp.bfloat16)
        z = (lax.dot_general(hi, eye, (((0,), (0,)), ((), ())),
                             preferred_element_type=jnp.float32)
             + lax.dot_general(lo, eye, (((0,), (0,)), ((), ())),
                               preferred_element_type=jnp.float32))
        out_ref[:, EMBED_DIM * s:EMBED_DIM * (s + 1)] = z


def _transpose_call(emb_t):
    return pl.pallas_call(
        _tr_body,
        grid=(TBLK,),
        in_specs=[pl.BlockSpec((16, TW), lambda i: (0, i))],
        out_specs=pl.BlockSpec((TC_, 8 * EMBED_DIM), lambda i: (i, 0)),
        out_shape=jax.ShapeDtypeStruct((PAD_PACKED, 8 * EMBED_DIM), jnp.float32),
    )(emb_t)


TILE = 1024


def _dense_body(dense_ref, emb_ref, bw0, bb0, bw1, bb1, bw2, bb2,
                v_ref, u_ref, cb_ref, tw0, tb0, tw1, tb1, tw2, tb2, out_ref):
    def dot(a, b):
        return lax.dot_general(a, b, (((1,), (0,)), ((), ())),
                               preferred_element_type=jnp.float32)

    h = jnp.maximum(dot(dense_ref[...], bw0[...]) + bb0[...], 0.0)
    h = jnp.maximum(dot(h, bw1[...]) + bb1[...], 0.0)
    h = jnp.maximum(dot(h, bw2[...]) + bb2[...], 0.0)
    x0 = jnp.concatenate([h, emb_ref[...]], axis=-1)
    xl = x0
    for i in range(3):
        proj = dot(dot(xl, v_ref[i]), u_ref[i]) + cb_ref[i]
        xl = x0 * proj + xl
    t = jnp.maximum(dot(xl, tw0[...]) + tb0[...], 0.0)
    t = jnp.maximum(dot(t, tw1[...]) + tb1[...], 0.0)
    z = dot(t, tw2[...]) + tb2[...]
    out_ref[...] = 1.0 / (1.0 + jnp.exp(-z))


def _im_tile(i):
    return (i, 0)


def _im_full(i):
    return (0, 0)


def _im_full3(i):
    return (0, 0, 0)


def _dense_call(dense, emb, bw0, bb0, bw1, bb1, bw2, bb2, v, u, cb,
                tw0, tb0, tw1, tb1, tw2, tb2):
    return pl.pallas_call(
        _dense_body,
        grid=(B // TILE,),
        in_specs=[
            pl.BlockSpec((TILE, DENSE_DIM), _im_tile),
            pl.BlockSpec((TILE, NUM_FIELDS * EMBED_DIM), _im_tile),
            pl.BlockSpec((DENSE_DIM, 512), _im_full),
            pl.BlockSpec((1, 512), _im_full),
            pl.BlockSpec((512, 256), _im_full),
            pl.BlockSpec((1, 256), _im_full),
            pl.BlockSpec((256, 16), _im_full),
            pl.BlockSpec((1, 16), _im_full),
            pl.BlockSpec((3, X0_DIM, 64), _im_full3),
            pl.BlockSpec((3, 64, X0_DIM), _im_full3),
            pl.BlockSpec((3, 1, X0_DIM), _im_full3),
            pl.BlockSpec((X0_DIM, 512), _im_full),
            pl.BlockSpec((1, 512), _im_full),
            pl.BlockSpec((512, 256), _im_full),
            pl.BlockSpec((1, 256), _im_full),
            pl.BlockSpec((256, 1), _im_full),
            pl.BlockSpec((1, 1), _im_full),
        ],
        out_specs=pl.BlockSpec((TILE, 1), _im_tile),
        out_shape=jax.ShapeDtypeStruct((B, 1), jnp.float32),
    )(dense, emb, bw0, bb0, bw1, bb1, bw2, bb2, v, u, cb,
      tw0, tb0, tw1, tb1, tw2, tb2)


def kernel(dense_features, preprocessed_sparse_features, emb_table,
           bw0, bb0, bw1, bb1, bw2, bb2,
           cross_V, cross_U, cross_b,
           tw0, tb0, tw1, tb1, tw2, tb2):
    offsets = (jnp.arange(NUM_FIELDS, dtype=jnp.int32) * VOCAB)[None, :]
    flat_idx = (preprocessed_sparse_features.astype(jnp.int32) + offsets)
    blk = flat_idx // TW
    o = flat_idx % TW
    pidx = (blk * TC_ + o % TC_) * 8 + o // TC_
    idx3 = pidx.reshape(NW, CHUNKS, CHUNK)
    emb_rm = _transpose_call(emb_table.T).reshape(PAD_ROWS, EMBED_DIM)
    emb_rows = _sc_gather(idx3, emb_rm)
    emb = emb_rows.reshape(B, NUM_FIELDS * EMBED_DIM)
    return _dense_call(
        dense_features, emb,
        bw0, bb0.reshape(1, -1), bw1, bb1.reshape(1, -1), bw2, bb2.reshape(1, -1),
        cross_V, cross_U, cross_b.reshape(3, 1, X0_DIM),
        tw0, tb0.reshape(1, -1), tw1, tb1.reshape(1, -1), tw2, tb2.reshape(1, -1))


# speedup vs baseline: 1.2574x; 1.2574x over previous
import functools

import jax
import jax.numpy as jnp
from jax import lax
from jax.experimental import pallas as pl
from jax.experimental.pallas import tpu as pltpu
from jax.experimental.pallas import tpu_sc as plsc

B = 16384
NUM_FIELDS = 26
VOCAB = 100000
EMBED_DIM = 16
DENSE_DIM = 13
X0_DIM = 432
N_LOOKUPS = B * NUM_FIELDS

NC = 2
NS = 16
NW = NC * NS
PER_W = N_LOOKUPS // NW
CHUNK = 128
CHUNKS = PER_W // CHUNK
G = 8
STEPS = CHUNKS // G


def _gather_body(idx_hbm, table_hbm, out_hbm, idx_v, rows_v, sem):
    wid = lax.axis_index("s") * NC + lax.axis_index("c")
    pltpu.sync_copy(idx_hbm.at[wid], idx_v)
    base = wid * PER_W

    def step(t, carry):
        cps = []
        for g in range(G):
            cps.append(pltpu.async_copy(
                table_hbm.at[idx_v.at[t * G + g]],
                rows_v.at[pl.ds(g * CHUNK, CHUNK)],
                sem))
        for cp in cps:
            cp.wait()
        pltpu.sync_copy(rows_v, out_hbm.at[pl.ds(base + t * (G * CHUNK), G * CHUNK)])
        return carry

    lax.fori_loop(0, STEPS, step, 0)


def _sc_gather(idx3, table):
    k = functools.partial(
        pl.kernel,
        mesh=plsc.VectorSubcoreMesh(core_axis_name="c", subcore_axis_name="s"),
        out_type=jax.ShapeDtypeStruct((N_LOOKUPS, EMBED_DIM), jnp.float32),
        scratch_types=[
            pltpu.VMEM((CHUNKS, CHUNK), jnp.int32),
            pltpu.VMEM((G * CHUNK, EMBED_DIM), jnp.float32),
            pltpu.SemaphoreType.DMA,
        ],
        compiler_params=pltpu.CompilerParams(use_tc_tiling_on_sc=False),
    )(_gather_body)
    return k(idx3, table)


N_ROWS = NUM_FIELDS * VOCAB
TW = 16384
TBLK = pl.cdiv(N_ROWS, TW)
TC_ = TW // 8
PAD_PACKED = TBLK * TC_
PAD_ROWS = PAD_PACKED * 8


def _tr_body(in_ref, out_ref):
    eye = jnp.eye(EMBED_DIM, dtype=jnp.bfloat16)
    c = TW // 8
    for s in range(8):
        z = lax.dot_general(in_ref[:, s * c:(s + 1) * c].astype(jnp.bfloat16),
                            eye, (((0,), (0,)), ((), ())),
                            preferred_element_type=jnp.float32)
        out_ref[:, EMBED_DIM * s:EMBED_DIM * (s + 1)] = z


def _transpose_call(emb_t):
    return pl.pallas_call(
        _tr_body,
        grid=(TBLK,),
        in_specs=[pl.BlockSpec((16, TW), lambda i: (0, i))],
        out_specs=pl.BlockSpec((TC_, 8 * EMBED_DIM), lambda i: (i, 0)),
        out_shape=jax.ShapeDtypeStruct((PAD_PACKED, 8 * EMBED_DIM), jnp.float32),
    )(emb_t)


TILE = 1024


def _dense_body(dense_ref, emb_ref, bw0, bb0, bw1, bb1, bw2, bb2,
                v_ref, u_ref, cb_ref, tw0, tb0, tw1, tb1, tw2, tb2, out_ref):
    def dot(a, b):
        return lax.dot_general(a, b, (((1,), (0,)), ((), ())),
                               preferred_element_type=jnp.float32)

    h = jnp.maximum(dot(dense_ref[...], bw0[...]) + bb0[...], 0.0)
    h = jnp.maximum(dot(h, bw1[...]) + bb1[...], 0.0)
    h = jnp.maximum(dot(h, bw2[...]) + bb2[...], 0.0)
    x0 = jnp.concatenate([h, emb_ref[...]], axis=-1)
    xl = x0
    for i in range(3):
        proj = dot(dot(xl, v_ref[i]), u_ref[i]) + cb_ref[i]
        xl = x0 * proj + xl
    t = jnp.maximum(dot(xl, tw0[...]) + tb0[...], 0.0)
    t = jnp.maximum(dot(t, tw1[...]) + tb1[...], 0.0)
    z = dot(t, tw2[...]) + tb2[...]
    out_ref[...] = 1.0 / (1.0 + jnp.exp(-z))


def _im_tile(i):
    return (i, 0)


def _im_full(i):
    return (0, 0)


def _im_full3(i):
    return (0, 0, 0)


def _dense_call(dense, emb, bw0, bb0, bw1, bb1, bw2, bb2, v, u, cb,
                tw0, tb0, tw1, tb1, tw2, tb2):
    return pl.pallas_call(
        _dense_body,
        grid=(B // TILE,),
        in_specs=[
            pl.BlockSpec((TILE, DENSE_DIM), _im_tile),
            pl.BlockSpec((TILE, NUM_FIELDS * EMBED_DIM), _im_tile),
            pl.BlockSpec((DENSE_DIM, 512), _im_full),
            pl.BlockSpec((1, 512), _im_full),
            pl.BlockSpec((512, 256), _im_full),
            pl.BlockSpec((1, 256), _im_full),
            pl.BlockSpec((256, 16), _im_full),
            pl.BlockSpec((1, 16), _im_full),
            pl.BlockSpec((3, X0_DIM, 64), _im_full3),
            pl.BlockSpec((3, 64, X0_DIM), _im_full3),
            pl.BlockSpec((3, 1, X0_DIM), _im_full3),
            pl.BlockSpec((X0_DIM, 512), _im_full),
            pl.BlockSpec((1, 512), _im_full),
            pl.BlockSpec((512, 256), _im_full),
            pl.BlockSpec((1, 256), _im_full),
            pl.BlockSpec((256, 1), _im_full),
            pl.BlockSpec((1, 1), _im_full),
        ],
        out_specs=pl.BlockSpec((TILE, 1), _im_tile),
        out_shape=jax.ShapeDtypeStruct((B, 1), jnp.float32),
    )(dense, emb, bw0, bb0, bw1, bb1, bw2, bb2, v, u, cb,
      tw0, tb0, tw1, tb1, tw2, tb2)


def kernel(dense_features, preprocessed_sparse_features, emb_table,
           bw0, bb0, bw1, bb1, bw2, bb2,
           cross_V, cross_U, cross_b,
           tw0, tb0, tw1, tb1, tw2, tb2):
    offsets = (jnp.arange(NUM_FIELDS, dtype=jnp.int32) * VOCAB)[None, :]
    flat_idx = (preprocessed_sparse_features.astype(jnp.int32) + offsets)
    blk = flat_idx // TW
    o = flat_idx % TW
    pidx = (blk * TC_ + o % TC_) * 8 + o // TC_
    idx3 = pidx.reshape(NW, CHUNKS, CHUNK)
    emb_rm = _transpose_call(emb_table.T).reshape(PAD_ROWS, EMBED_DIM)
    emb_rows = _sc_gather(idx3, emb_rm)
    emb = emb_rows.reshape(B, NUM_FIELDS * EMBED_DIM)
    return _dense_call(
        dense_features, emb,
        bw0, bb0.reshape(1, -1), bw1, bb1.reshape(1, -1), bw2, bb2.reshape(1, -1),
        cross_V, cross_U, cross_b.reshape(3, 1, X0_DIM),
        tw0, tb0.reshape(1, -1), tw1, tb1.reshape(1, -1), tw2, tb2.reshape(1, -1))


# speedup vs baseline: 1.9904x; 1.5829x over previous
import functools

import jax
import jax.numpy as jnp
from jax import lax
from jax.experimental import pallas as pl
from jax.experimental.pallas import tpu as pltpu
from jax.experimental.pallas import tpu_sc as plsc

B = 16384
NUM_FIELDS = 26
VOCAB = 100000
EMBED_DIM = 16
DENSE_DIM = 13
X0_DIM = 432
N_LOOKUPS = B * NUM_FIELDS

NC = 2
NS = 16
NW = NC * NS
PER_W = N_LOOKUPS // NW
CHUNK = 128
CHUNKS = PER_W // CHUNK
G = 8
STEPS = CHUNKS // G


def _gather_body(idx_hbm, table_hbm, out_hbm, idx_v, rows_v, sem):
    wid = lax.axis_index("s") * NC + lax.axis_index("c")
    pltpu.sync_copy(idx_hbm.at[wid], idx_v)
    base = wid * PER_W

    def step(t, carry):
        cps = []
        for g in range(G):
            cps.append(pltpu.async_copy(
                table_hbm.at[idx_v.at[t * G + g]],
                rows_v.at[pl.ds(g * CHUNK, CHUNK)],
                sem))
        for cp in cps:
            cp.wait()
        pltpu.sync_copy(rows_v, out_hbm.at[pl.ds(base + t * (G * CHUNK), G * CHUNK)])
        return carry

    lax.fori_loop(0, STEPS, step, 0)


def _sc_gather(idx3, table):
    k = functools.partial(
        pl.kernel,
        mesh=plsc.VectorSubcoreMesh(core_axis_name="c", subcore_axis_name="s"),
        out_type=jax.ShapeDtypeStruct((N_LOOKUPS, EMBED_DIM), jnp.float32),
        scratch_types=[
            pltpu.VMEM((CHUNKS, CHUNK), jnp.int32),
            pltpu.VMEM((G * CHUNK, EMBED_DIM), jnp.float32),
            pltpu.SemaphoreType.DMA,
        ],
        compiler_params=pltpu.CompilerParams(use_tc_tiling_on_sc=False),
    )(_gather_body)
    return k(idx3, table)


N_ROWS = NUM_FIELDS * VOCAB
TW = 16384
TBLK = pl.cdiv(N_ROWS, TW)
TC_ = TW // 8
PAD_PACKED = TBLK * TC_
PAD_ROWS = PAD_PACKED * 8


def _tr_body(in_ref, out_ref):
    c = TW // 8
    base = jax.lax.broadcasted_iota(jnp.int32, (EMBED_DIM, 8 * EMBED_DIM), 1)
    row = jax.lax.broadcasted_iota(jnp.int32, (EMBED_DIM, 8 * EMBED_DIM), 0)
    acc = None
    for s in range(8):
        sel = (base == row + EMBED_DIM * s).astype(jnp.bfloat16)
        z = lax.dot_general(in_ref[:, s * c:(s + 1) * c].astype(jnp.bfloat16),
                            sel, (((0,), (0,)), ((), ())),
                            preferred_element_type=jnp.float32)
        acc = z if acc is None else acc + z
    out_ref[...] = acc


def _transpose_call(emb_t):
    return pl.pallas_call(
        _tr_body,
        grid=(TBLK,),
        in_specs=[pl.BlockSpec((16, TW), lambda i: (0, i))],
        out_specs=pl.BlockSpec((TC_, 8 * EMBED_DIM), lambda i: (i, 0)),
        out_shape=jax.ShapeDtypeStruct((PAD_PACKED, 8 * EMBED_DIM), jnp.float32),
    )(emb_t)


TILE = 1024


def _dense_body(dense_ref, emb_ref, bw0, bb0, bw1, bb1, bw2, bb2,
                v_ref, u_ref, cb_ref, tw0, tb0, tw1, tb1, tw2, tb2, out_ref):
    def dot(a, b):
        return lax.dot_general(a, b, (((1,), (0,)), ((), ())),
                               preferred_element_type=jnp.float32)

    h = jnp.maximum(dot(dense_ref[...], bw0[...]) + bb0[...], 0.0)
    h = jnp.maximum(dot(h, bw1[...]) + bb1[...], 0.0)
    h = jnp.maximum(dot(h, bw2[...]) + bb2[...], 0.0)
    x0 = jnp.concatenate([h, emb_ref[...]], axis=-1)
    xl = x0
    for i in range(3):
        proj = dot(dot(xl, v_ref[i]), u_ref[i]) + cb_ref[i]
        xl = x0 * proj + xl
    t = jnp.maximum(dot(xl, tw0[...]) + tb0[...], 0.0)
    t = jnp.maximum(dot(t, tw1[...]) + tb1[...], 0.0)
    z = dot(t, tw2[...]) + tb2[...]
    out_ref[...] = 1.0 / (1.0 + jnp.exp(-z))


def _im_tile(i):
    return (i, 0)


def _im_full(i):
    return (0, 0)


def _im_full3(i):
    return (0, 0, 0)


def _dense_call(dense, emb, bw0, bb0, bw1, bb1, bw2, bb2, v, u, cb,
                tw0, tb0, tw1, tb1, tw2, tb2):
    return pl.pallas_call(
        _dense_body,
        grid=(B // TILE,),
        in_specs=[
            pl.BlockSpec((TILE, DENSE_DIM), _im_tile),
            pl.BlockSpec((TILE, NUM_FIELDS * EMBED_DIM), _im_tile),
            pl.BlockSpec((DENSE_DIM, 512), _im_full),
            pl.BlockSpec((1, 512), _im_full),
            pl.BlockSpec((512, 256), _im_full),
            pl.BlockSpec((1, 256), _im_full),
            pl.BlockSpec((256, 16), _im_full),
            pl.BlockSpec((1, 16), _im_full),
            pl.BlockSpec((3, X0_DIM, 64), _im_full3),
            pl.BlockSpec((3, 64, X0_DIM), _im_full3),
            pl.BlockSpec((3, 1, X0_DIM), _im_full3),
            pl.BlockSpec((X0_DIM, 512), _im_full),
            pl.BlockSpec((1, 512), _im_full),
            pl.BlockSpec((512, 256), _im_full),
            pl.BlockSpec((1, 256), _im_full),
            pl.BlockSpec((256, 1), _im_full),
            pl.BlockSpec((1, 1), _im_full),
        ],
        out_specs=pl.BlockSpec((TILE, 1), _im_tile),
        out_shape=jax.ShapeDtypeStruct((B, 1), jnp.float32),
    )(dense, emb, bw0, bb0, bw1, bb1, bw2, bb2, v, u, cb,
      tw0, tb0, tw1, tb1, tw2, tb2)


def kernel(dense_features, preprocessed_sparse_features, emb_table,
           bw0, bb0, bw1, bb1, bw2, bb2,
           cross_V, cross_U, cross_b,
           tw0, tb0, tw1, tb1, tw2, tb2):
    offsets = (jnp.arange(NUM_FIELDS, dtype=jnp.int32) * VOCAB)[None, :]
    flat_idx = (preprocessed_sparse_features.astype(jnp.int32) + offsets)
    blk = flat_idx // TW
    o = flat_idx % TW
    pidx = (blk * TC_ + o % TC_) * 8 + o // TC_
    idx3 = pidx.reshape(NW, CHUNKS, CHUNK)
    emb_rm = _transpose_call(emb_table.T).reshape(PAD_ROWS, EMBED_DIM)
    emb_rows = _sc_gather(idx3, emb_rm)
    emb = emb_rows.reshape(B, NUM_FIELDS * EMBED_DIM)
    return _dense_call(
        dense_features, emb,
        bw0, bb0.reshape(1, -1), bw1, bb1.reshape(1, -1), bw2, bb2.reshape(1, -1),
        cross_V, cross_U, cross_b.reshape(3, 1, X0_DIM),
        tw0, tb0.reshape(1, -1), tw1, tb1.reshape(1, -1), tw2, tb2.reshape(1, -1))


# speedup vs baseline: 2.2213x; 1.1160x over previous
import functools

import jax
import jax.numpy as jnp
from jax import lax
from jax.experimental import pallas as pl
from jax.experimental.pallas import tpu as pltpu
from jax.experimental.pallas import tpu_sc as plsc

B = 16384
NUM_FIELDS = 26
VOCAB = 100000
EMBED_DIM = 16
DENSE_DIM = 13
X0_DIM = 432
N_LOOKUPS = B * NUM_FIELDS

NC = 2
NS = 16
NW = NC * NS
PER_W = N_LOOKUPS // NW
CHUNK = 128
CHUNKS = PER_W // CHUNK
G = 8
STEPS = CHUNKS // G


def _gather_body(idx_hbm, table_hbm, out_hbm, idx_v, rows_v, sem):
    wid = lax.axis_index("s") * NC + lax.axis_index("c")
    pltpu.sync_copy(idx_hbm.at[wid], idx_v)
    base = wid * PER_W

    def step(t, carry):
        cps = []
        for g in range(G):
            cps.append(pltpu.async_copy(
                table_hbm.at[idx_v.at[t * G + g]],
                rows_v.at[pl.ds(g * CHUNK, CHUNK)],
                sem))
        for cp in cps:
            cp.wait()
        pltpu.sync_copy(rows_v, out_hbm.at[pl.ds(base + t * (G * CHUNK), G * CHUNK)])
        return carry

    lax.fori_loop(0, STEPS, step, 0)


def _sc_gather(idx3, table):
    k = functools.partial(
        pl.kernel,
        mesh=plsc.VectorSubcoreMesh(core_axis_name="c", subcore_axis_name="s"),
        out_type=jax.ShapeDtypeStruct((N_LOOKUPS, EMBED_DIM), jnp.float32),
        scratch_types=[
            pltpu.VMEM((CHUNKS, CHUNK), jnp.int32),
            pltpu.VMEM((G * CHUNK, EMBED_DIM), jnp.float32),
            pltpu.SemaphoreType.DMA,
        ],
        compiler_params=pltpu.CompilerParams(use_tc_tiling_on_sc=False),
    )(_gather_body)
    return k(idx3, table)


N_ROWS = NUM_FIELDS * VOCAB
TW = 32768
TBLK = pl.cdiv(N_ROWS, TW)
TC_ = TW // 8
PAD_PACKED = TBLK * TC_
PAD_ROWS = PAD_PACKED * 8


def _tr_body(in_ref, out_ref):
    c = TW // 8
    base = jax.lax.broadcasted_iota(jnp.int32, (EMBED_DIM, 8 * EMBED_DIM), 1)
    row = jax.lax.broadcasted_iota(jnp.int32, (EMBED_DIM, 8 * EMBED_DIM), 0)
    acc = None
    for s in range(8):
        sel = (base == row + EMBED_DIM * s).astype(jnp.bfloat16)
        z = lax.dot_general(in_ref[:, s * c:(s + 1) * c].astype(jnp.bfloat16),
                            sel, (((0,), (0,)), ((), ())),
                            preferred_element_type=jnp.float32)
        acc = z if acc is None else acc + z
    out_ref[...] = acc


def _transpose_call(emb_t):
    return pl.pallas_call(
        _tr_body,
        grid=(TBLK,),
        in_specs=[pl.BlockSpec((16, TW), lambda i: (0, i))],
        out_specs=pl.BlockSpec((TC_, 8 * EMBED_DIM), lambda i: (i, 0)),
        out_shape=jax.ShapeDtypeStruct((PAD_PACKED, 8 * EMBED_DIM), jnp.float32),
    )(emb_t)


TILE = 1024


def _dense_body(dense_ref, emb_ref, bw0, bb0, bw1, bb1, bw2, bb2,
                v_ref, u_ref, cb_ref, tw0, tb0, tw1, tb1, tw2, tb2, out_ref):
    def dot(a, b):
        return lax.dot_general(a, b, (((1,), (0,)), ((), ())),
                               preferred_element_type=jnp.float32)

    h = jnp.maximum(dot(dense_ref[...], bw0[...]) + bb0[...], 0.0)
    h = jnp.maximum(dot(h, bw1[...]) + bb1[...], 0.0)
    h = jnp.maximum(dot(h, bw2[...]) + bb2[...], 0.0)
    x0 = jnp.concatenate([h, emb_ref[...]], axis=-1)
    xl = x0
    for i in range(3):
        proj = dot(dot(xl, v_ref[i]), u_ref[i]) + cb_ref[i]
        xl = x0 * proj + xl
    t = jnp.maximum(dot(xl, tw0[...]) + tb0[...], 0.0)
    t = jnp.maximum(dot(t, tw1[...]) + tb1[...], 0.0)
    z = dot(t, tw2[...]) + tb2[...]
    out_ref[...] = 1.0 / (1.0 + jnp.exp(-z))


def _im_tile(i):
    return (i, 0)


def _im_full(i):
    return (0, 0)


def _im_full3(i):
    return (0, 0, 0)


def _dense_call(dense, emb, bw0, bb0, bw1, bb1, bw2, bb2, v, u, cb,
                tw0, tb0, tw1, tb1, tw2, tb2):
    return pl.pallas_call(
        _dense_body,
        grid=(B // TILE,),
        in_specs=[
            pl.BlockSpec((TILE, DENSE_DIM), _im_tile),
            pl.BlockSpec((TILE, NUM_FIELDS * EMBED_DIM), _im_tile),
            pl.BlockSpec((DENSE_DIM, 512), _im_full),
            pl.BlockSpec((1, 512), _im_full),
            pl.BlockSpec((512, 256), _im_full),
            pl.BlockSpec((1, 256), _im_full),
            pl.BlockSpec((256, 16), _im_full),
            pl.BlockSpec((1, 16), _im_full),
            pl.BlockSpec((3, X0_DIM, 64), _im_full3),
            pl.BlockSpec((3, 64, X0_DIM), _im_full3),
            pl.BlockSpec((3, 1, X0_DIM), _im_full3),
            pl.BlockSpec((X0_DIM, 512), _im_full),
            pl.BlockSpec((1, 512), _im_full),
            pl.BlockSpec((512, 256), _im_full),
            pl.BlockSpec((1, 256), _im_full),
            pl.BlockSpec((256, 1), _im_full),
            pl.BlockSpec((1, 1), _im_full),
        ],
        out_specs=pl.BlockSpec((TILE, 1), _im_tile),
        out_shape=jax.ShapeDtypeStruct((B, 1), jnp.float32),
    )(dense, emb, bw0, bb0, bw1, bb1, bw2, bb2, v, u, cb,
      tw0, tb0, tw1, tb1, tw2, tb2)


def kernel(dense_features, preprocessed_sparse_features, emb_table,
           bw0, bb0, bw1, bb1, bw2, bb2,
           cross_V, cross_U, cross_b,
           tw0, tb0, tw1, tb1, tw2, tb2):
    offsets = (jnp.arange(NUM_FIELDS, dtype=jnp.int32) * VOCAB)[None, :]
    flat_idx = (preprocessed_sparse_features.astype(jnp.int32) + offsets)
    blk = flat_idx // TW
    o = flat_idx % TW
    pidx = (blk * TC_ + o % TC_) * 8 + o // TC_
    idx3 = pidx.reshape(NW, CHUNKS, CHUNK)
    emb_rm = _transpose_call(emb_table.T).reshape(PAD_ROWS, EMBED_DIM)
    emb_rows = _sc_gather(idx3, emb_rm)
    emb = emb_rows.reshape(B, NUM_FIELDS * EMBED_DIM)
    return _dense_call(
        dense_features, emb,
        bw0, bb0.reshape(1, -1), bw1, bb1.reshape(1, -1), bw2, bb2.reshape(1, -1),
        cross_V, cross_U, cross_b.reshape(3, 1, X0_DIM),
        tw0, tb0.reshape(1, -1), tw1, tb1.reshape(1, -1), tw2, tb2.reshape(1, -1))


# speedup vs baseline: 2.2799x; 1.0264x over previous
import functools

import jax
import jax.numpy as jnp
from jax import lax
from jax.experimental import pallas as pl
from jax.experimental.pallas import tpu as pltpu
from jax.experimental.pallas import tpu_sc as plsc

B = 16384
NUM_FIELDS = 26
VOCAB = 100000
EMBED_DIM = 16
DENSE_DIM = 13
X0_DIM = 432
N_LOOKUPS = B * NUM_FIELDS

NC = 2
NS = 16
NW = NC * NS
PER_W = N_LOOKUPS // NW
CHUNK = 128
CHUNKS = PER_W // CHUNK
G = 8
STEPS = CHUNKS // G


def _gather_body(idx_hbm, table_hbm, out_hbm, idx_v, rows_v, sem):
    wid = lax.axis_index("s") * NC + lax.axis_index("c")
    pltpu.sync_copy(idx_hbm.at[wid], idx_v)
    base = wid * PER_W

    def step(t, carry):
        cps = []
        for g in range(G):
            cps.append(pltpu.async_copy(
                table_hbm.at[idx_v.at[t * G + g]],
                rows_v.at[pl.ds(g * CHUNK, CHUNK)],
                sem))
        for cp in cps:
            cp.wait()
        pltpu.sync_copy(rows_v, out_hbm.at[pl.ds(base + t * (G * CHUNK), G * CHUNK)])
        return carry

    lax.fori_loop(0, STEPS, step, 0)


def _sc_gather(idx3, table):
    k = functools.partial(
        pl.kernel,
        mesh=plsc.VectorSubcoreMesh(core_axis_name="c", subcore_axis_name="s"),
        out_type=jax.ShapeDtypeStruct((N_LOOKUPS, EMBED_DIM), jnp.float32),
        scratch_types=[
            pltpu.VMEM((CHUNKS, CHUNK), jnp.int32),
            pltpu.VMEM((G * CHUNK, EMBED_DIM), jnp.float32),
            pltpu.SemaphoreType.DMA,
        ],
        compiler_params=pltpu.CompilerParams(use_tc_tiling_on_sc=False),
    )(_gather_body)
    return k(idx3, table)


N_ROWS = NUM_FIELDS * VOCAB
TW = 65536
TBLK = pl.cdiv(N_ROWS, TW)
TC_ = TW // 8
PAD_PACKED = TBLK * TC_
PAD_ROWS = PAD_PACKED * 8


def _tr_body(in_ref, out_ref):
    c = TW // 8
    base = jax.lax.broadcasted_iota(jnp.int32, (EMBED_DIM, 8 * EMBED_DIM), 1)
    row = jax.lax.broadcasted_iota(jnp.int32, (EMBED_DIM, 8 * EMBED_DIM), 0)
    acc = None
    for s in range(8):
        sel = (base == row + EMBED_DIM * s).astype(jnp.bfloat16)
        z = lax.dot_general(in_ref[:, s * c:(s + 1) * c].astype(jnp.bfloat16),
                            sel, (((0,), (0,)), ((), ())),
                            preferred_element_type=jnp.float32)
        acc = z if acc is None else acc + z
    out_ref[...] = acc


def _transpose_call(emb_t):
    return pl.pallas_call(
        _tr_body,
        grid=(TBLK,),
        in_specs=[pl.BlockSpec((16, TW), lambda i: (0, i))],
        out_specs=pl.BlockSpec((TC_, 8 * EMBED_DIM), lambda i: (i, 0)),
        out_shape=jax.ShapeDtypeStruct((PAD_PACKED, 8 * EMBED_DIM), jnp.float32),
    )(emb_t)


TILE = 1024


def _dense_body(dense_ref, emb_ref, bw0, bb0, bw1, bb1, bw2, bb2,
                v_ref, u_ref, cb_ref, tw0, tb0, tw1, tb1, tw2, tb2, out_ref):
    def dot(a, b):
        return lax.dot_general(a, b, (((1,), (0,)), ((), ())),
                               preferred_element_type=jnp.float32)

    h = jnp.maximum(dot(dense_ref[...], bw0[...]) + bb0[...], 0.0)
    h = jnp.maximum(dot(h, bw1[...]) + bb1[...], 0.0)
    h = jnp.maximum(dot(h, bw2[...]) + bb2[...], 0.0)
    x0 = jnp.concatenate([h, emb_ref[...]], axis=-1)
    xl = x0
    for i in range(3):
        proj = dot(dot(xl, v_ref[i]), u_ref[i]) + cb_ref[i]
        xl = x0 * proj + xl
    t = jnp.maximum(dot(xl, tw0[...]) + tb0[...], 0.0)
    t = jnp.maximum(dot(t, tw1[...]) + tb1[...], 0.0)
    z = dot(t, tw2[...]) + tb2[...]
    out_ref[...] = 1.0 / (1.0 + jnp.exp(-z))


def _im_tile(i):
    return (i, 0)


def _im_full(i):
    return (0, 0)


def _im_full3(i):
    return (0, 0, 0)


def _dense_call(dense, emb, bw0, bb0, bw1, bb1, bw2, bb2, v, u, cb,
                tw0, tb0, tw1, tb1, tw2, tb2):
    return pl.pallas_call(
        _dense_body,
        grid=(B // TILE,),
        in_specs=[
            pl.BlockSpec((TILE, DENSE_DIM), _im_tile),
            pl.BlockSpec((TILE, NUM_FIELDS * EMBED_DIM), _im_tile),
            pl.BlockSpec((DENSE_DIM, 512), _im_full),
            pl.BlockSpec((1, 512), _im_full),
            pl.BlockSpec((512, 256), _im_full),
            pl.BlockSpec((1, 256), _im_full),
            pl.BlockSpec((256, 16), _im_full),
            pl.BlockSpec((1, 16), _im_full),
            pl.BlockSpec((3, X0_DIM, 64), _im_full3),
            pl.BlockSpec((3, 64, X0_DIM), _im_full3),
            pl.BlockSpec((3, 1, X0_DIM), _im_full3),
            pl.BlockSpec((X0_DIM, 512), _im_full),
            pl.BlockSpec((1, 512), _im_full),
            pl.BlockSpec((512, 256), _im_full),
            pl.BlockSpec((1, 256), _im_full),
            pl.BlockSpec((256, 1), _im_full),
            pl.BlockSpec((1, 1), _im_full),
        ],
        out_specs=pl.BlockSpec((TILE, 1), _im_tile),
        out_shape=jax.ShapeDtypeStruct((B, 1), jnp.float32),
    )(dense, emb, bw0, bb0, bw1, bb1, bw2, bb2, v, u, cb,
      tw0, tb0, tw1, tb1, tw2, tb2)


def kernel(dense_features, preprocessed_sparse_features, emb_table,
           bw0, bb0, bw1, bb1, bw2, bb2,
           cross_V, cross_U, cross_b,
           tw0, tb0, tw1, tb1, tw2, tb2):
    offsets = (jnp.arange(NUM_FIELDS, dtype=jnp.int32) * VOCAB)[None, :]
    flat_idx = (preprocessed_sparse_features.astype(jnp.int32) + offsets)
    blk = flat_idx // TW
    o = flat_idx % TW
    pidx = (blk * TC_ + o % TC_) * 8 + o // TC_
    idx3 = pidx.reshape(NW, CHUNKS, CHUNK)
    emb_rm = _transpose_call(emb_table.T).reshape(PAD_ROWS, EMBED_DIM)
    emb_rows = _sc_gather(idx3, emb_rm)
    emb = emb_rows.reshape(B, NUM_FIELDS * EMBED_DIM)
    return _dense_call(
        dense_features, emb,
        bw0, bb0.reshape(1, -1), bw1, bb1.reshape(1, -1), bw2, bb2.reshape(1, -1),
        cross_V, cross_U, cross_b.reshape(3, 1, X0_DIM),
        tw0, tb0.reshape(1, -1), tw1, tb1.reshape(1, -1), tw2, tb2.reshape(1, -1))
